# baseline (speedup 1.0000x reference)
import functools

import jax
import jax.numpy as jnp
from jax import lax
from jax.experimental import pallas as pl
from jax.experimental.pallas import tpu as pltpu
from jax.experimental.pallas import tpu_sc as plsc

_NUM_CODES = 8192
_DIM = 256
_TOKENS = 16384
_MBLK = 512
_COMMIT = 0.25


def _argmin_body(z_ref, w_ref, zn_ref, cn_ref, iota_ref, idx_ref, dsum_ref):
    z = z_ref[...]
    w = w_ref[...]
    zn = zn_ref[...]
    cn = cn_ref[...]
    mm2 = lax.dot_general(z.astype(jnp.bfloat16) * jnp.bfloat16(2.0),
                          w.astype(jnp.bfloat16),
                          (((1,), (1,)), ((), ())),
                          preferred_element_type=jnp.float32)
    dist = (zn + cn) - mm2
    dminv = jnp.min(dist, axis=1, keepdims=True)
    idx_ref[...] = jnp.min(
        jnp.where(dist == dminv, iota_ref[...], _NUM_CODES), axis=1)
    dmin = dminv[:, 0]

    @pl.when(pl.program_id(0) == 0)
    def _():
        dsum_ref[0, 0] = 0.0

    dsum_ref[0, 0] += jnp.sum(dmin)


def _distance_argmin(z_e, W, zn, cn):
    grid = _TOKENS // _MBLK
    return pl.pallas_call(
        _argmin_body,
        grid=(grid,),
        in_specs=[
            pl.BlockSpec((_MBLK, _DIM), lambda i: (i, 0)),
            pl.BlockSpec((_NUM_CODES, _DIM), lambda i: (0, 0)),
            pl.BlockSpec((_MBLK, 1), lambda i: (i, 0)),
            pl.BlockSpec((1, _NUM_CODES), lambda i: (0, 0)),
            pl.BlockSpec((1, _NUM_CODES), lambda i: (0, 0)),
        ],
        out_specs=[
            pl.BlockSpec((_MBLK,), lambda i: (i,)),
            pl.BlockSpec(memory_space=pltpu.SMEM, block_shape=(1, 1),
                         index_map=lambda i: (0, 0)),
        ],
        out_shape=[
            jax.ShapeDtypeStruct((_TOKENS,), jnp.int32),
            jax.ShapeDtypeStruct((1, 1), jnp.float32),
        ],
        compiler_params=pltpu.CompilerParams(
            dimension_semantics=("arbitrary",),
            vmem_limit_bytes=100 * 1024 * 1024,
        ),
    )(z_e, W, zn, cn,
      lax.broadcasted_iota(jnp.int32, (1, _NUM_CODES), 1))


_SC_CHUNK = 128


def _sc_gather(W, idx):
    info = plsc.get_sparse_core_info()
    nc, ns = info.num_cores, info.num_subcores
    nw = nc * ns
    b_per_w = _TOKENS // nw
    n_chunks = b_per_w // _SC_CHUNK
    mesh = plsc.VectorSubcoreMesh(core_axis_name="c", subcore_axis_name="s")

    @functools.partial(
        pl.kernel,
        out_type=jax.ShapeDtypeStruct((_TOKENS, _DIM), jnp.float32),
        mesh=mesh,
        scratch_types=[
            pltpu.VMEM((_SC_CHUNK,), jnp.int32),
            pltpu.VMEM((_SC_CHUNK, _DIM), jnp.float32),
            pltpu.SemaphoreType.DMA,
        ],
    )
    def gather_k(table_hbm, idx_hbm, out_hbm, idx_v, rows_v, sem):
        wid = lax.axis_index("s") * nc + lax.axis_index("c")
        for g in range(n_chunks):
            base = wid * b_per_w + g * _SC_CHUNK
            pltpu.sync_copy(idx_hbm.at[pl.ds(base, _SC_CHUNK)], idx_v)
            pltpu.async_copy(table_hbm.at[idx_v], rows_v, sem).wait()
            pltpu.sync_copy(rows_v, out_hbm.at[pl.ds(base, _SC_CHUNK)])

    return gather_k(W, idx)


def kernel(z_e, W):
    zn = jnp.sum(z_e ** 2, axis=1, keepdims=True)
    cn = jnp.sum(W ** 2, axis=1).reshape(1, _NUM_CODES)
    idx, dsum = _distance_argmin(z_e, W, zn, cn)
    z_q_st = _sc_gather(W, idx)
    loss = (dsum[0, 0] / jnp.float32(_TOKENS * _DIM)) * jnp.float32(_COMMIT)
    return (z_q_st, idx, loss)


# speedup vs baseline: 1.1516x; 1.1516x over previous
import functools

import jax
import jax.numpy as jnp
from jax import lax
from jax.experimental import pallas as pl
from jax.experimental.pallas import tpu as pltpu
from jax.experimental.pallas import tpu_sc as plsc

_NUM_CODES = 8192
_DIM = 256
_TOKENS = 16384
_MBLK = 512
_COMMIT = 0.25


def _argmin_body(z_ref, w_ref, zn_ref, cn_ref, idx_ref, dsum_ref):
    z = z_ref[...]
    w = w_ref[...]
    zn = zn_ref[...]
    cn = cn_ref[0, :]
    mm = lax.dot_general(z, w, (((1,), (1,)), ((), ())),
                         preferred_element_type=jnp.float32)
    dist = zn + cn[None, :] - 2.0 * mm
    dminv = jnp.min(dist, axis=1, keepdims=True)
    iota = lax.broadcasted_iota(jnp.int32, dist.shape, 1)
    idx_ref[...] = jnp.min(
        jnp.where(dist == dminv, iota, _NUM_CODES), axis=1)
    dmin = dminv[:, 0]

    @pl.when(pl.program_id(0) == 0)
    def _():
        dsum_ref[0, 0] = 0.0

    dsum_ref[0, 0] += jnp.sum(dmin)


def _distance_argmin(z_e, W, zn, cn):
    grid = _TOKENS // _MBLK
    return pl.pallas_call(
        _argmin_body,
        grid=(grid,),
        in_specs=[
            pl.BlockSpec((_MBLK, _DIM), lambda i: (i, 0)),
            pl.BlockSpec((_NUM_CODES, _DIM), lambda i: (0, 0)),
            pl.BlockSpec((_MBLK, 1), lambda i: (i, 0)),
            pl.BlockSpec((1, _NUM_CODES), lambda i: (0, 0)),
        ],
        out_specs=[
            pl.BlockSpec((_MBLK,), lambda i: (i,)),
            pl.BlockSpec(memory_space=pltpu.SMEM, block_shape=(1, 1),
                         index_map=lambda i: (0, 0)),
        ],
        out_shape=[
            jax.ShapeDtypeStruct((_TOKENS,), jnp.int32),
            jax.ShapeDtypeStruct((1, 1), jnp.float32),
        ],
        compiler_params=pltpu.CompilerParams(
            dimension_semantics=("arbitrary",),
            vmem_limit_bytes=100 * 1024 * 1024,
        ),
    )(z_e.astype(jnp.bfloat16), W.astype(jnp.bfloat16), zn, cn)


_SC_CHUNK = 128


def _sc_gather(W, idx):
    info = plsc.get_sparse_core_info()
    nc, ns = info.num_cores, info.num_subcores
    nw = nc * ns
    b_per_w = _TOKENS // nw
    n_chunks = b_per_w // _SC_CHUNK
    mesh = plsc.VectorSubcoreMesh(core_axis_name="c", subcore_axis_name="s")

    @functools.partial(
        pl.kernel,
        out_type=jax.ShapeDtypeStruct((_TOKENS, _DIM), jnp.float32),
        mesh=mesh,
        scratch_types=[
            pltpu.VMEM((_SC_CHUNK,), jnp.int32),
            pltpu.VMEM((_SC_CHUNK, _DIM), jnp.float32),
            pltpu.SemaphoreType.DMA,
        ],
    )
    def gather_k(table_hbm, idx_hbm, out_hbm, idx_v, rows_v, sem):
        wid = lax.axis_index("s") * nc + lax.axis_index("c")
        for g in range(n_chunks):
            base = wid * b_per_w + g * _SC_CHUNK
            pltpu.sync_copy(idx_hbm.at[pl.ds(base, _SC_CHUNK)], idx_v)
            pltpu.async_copy(table_hbm.at[idx_v], rows_v, sem).wait()
            pltpu.sync_copy(rows_v, out_hbm.at[pl.ds(base, _SC_CHUNK)])

    return gather_k(W, idx)


def kernel(z_e, W):
    zn = jnp.sum(z_e ** 2, axis=1, keepdims=True)
    cn = jnp.sum(W ** 2, axis=1).reshape(1, _NUM_CODES)
    idx, dsum = _distance_argmin(z_e, W, zn, cn)
    z_q_st = _sc_gather(W, idx)
    loss = (dsum[0, 0] / jnp.float32(_TOKENS * _DIM)) * jnp.float32(_COMMIT)
    return (z_q_st, idx, loss)


# speedup vs baseline: 1.1830x; 1.0273x over previous
import functools

import jax
import jax.numpy as jnp
from jax import lax
from jax.experimental import pallas as pl
from jax.experimental.pallas import tpu as pltpu
from jax.experimental.pallas import tpu_sc as plsc

_NUM_CODES = 8192
_DIM = 256
_TOKENS = 16384
_MBLK = 512
_COMMIT = 0.25


def _argmin_body(z_ref, w_ref, zn_ref, cn_ref, idx_ref, dsum_ref):
    z = z_ref[...]
    w = w_ref[...]
    zn = zn_ref[...]
    cn = cn_ref[0, :]
    mm = lax.dot_general(z.astype(jnp.bfloat16), w, (((1,), (1,)), ((), ())),
                         preferred_element_type=jnp.float32)
    dist = zn + cn[None, :] - 2.0 * mm
    dminv = jnp.min(dist, axis=1, keepdims=True)
    iota = lax.broadcasted_iota(jnp.int32, dist.shape, 1)
    idx_ref[...] = jnp.min(
        jnp.where(dist == dminv, iota, _NUM_CODES), axis=1)
    dmin = dminv[:, 0]

    @pl.when(pl.program_id(0) == 0)
    def _():
        dsum_ref[0, 0] = 0.0

    dsum_ref[0, 0] += jnp.sum(dmin)


def _distance_argmin(z_e, W, zn, cn):
    grid = _TOKENS // _MBLK
    return pl.pallas_call(
        _argmin_body,
        grid=(grid,),
        in_specs=[
            pl.BlockSpec((_MBLK, _DIM), lambda i: (i, 0)),
            pl.BlockSpec((_NUM_CODES, _DIM), lambda i: (0, 0)),
            pl.BlockSpec((_MBLK, 1), lambda i: (i, 0)),
            pl.BlockSpec((1, _NUM_CODES), lambda i: (0, 0)),
        ],
        out_specs=[
            pl.BlockSpec((_MBLK,), lambda i: (i,)),
            pl.BlockSpec(memory_space=pltpu.SMEM, block_shape=(1, 1),
                         index_map=lambda i: (0, 0)),
        ],
        out_shape=[
            jax.ShapeDtypeStruct((_TOKENS,), jnp.int32),
            jax.ShapeDtypeStruct((1, 1), jnp.float32),
        ],
        compiler_params=pltpu.CompilerParams(
            dimension_semantics=("arbitrary",),
            vmem_limit_bytes=100 * 1024 * 1024,
        ),
    )(z_e, W.astype(jnp.bfloat16), zn, cn)


_SC_CHUNK = 128


def _sc_gather(W, idx):
    info = plsc.get_sparse_core_info()
    nc, ns = info.num_cores, info.num_subcores
    nw = nc * ns
    b_per_w = _TOKENS // nw
    n_chunks = b_per_w // _SC_CHUNK
    mesh = plsc.VectorSubcoreMesh(core_axis_name="c", subcore_axis_name="s")

    @functools.partial(
        pl.kernel,
        out_type=jax.ShapeDtypeStruct((_TOKENS, _DIM), jnp.float32),
        mesh=mesh,
        scratch_types=[
            pltpu.VMEM((_SC_CHUNK,), jnp.int32),
            pltpu.VMEM((_SC_CHUNK, _DIM), jnp.float32),
            pltpu.SemaphoreType.DMA,
        ],
    )
    def gather_k(table_hbm, idx_hbm, out_hbm, idx_v, rows_v, sem):
        wid = lax.axis_index("s") * nc + lax.axis_index("c")
        for g in range(n_chunks):
            base = wid * b_per_w + g * _SC_CHUNK
            pltpu.sync_copy(idx_hbm.at[pl.ds(base, _SC_CHUNK)], idx_v)
            pltpu.async_copy(table_hbm.at[idx_v], rows_v, sem).wait()
            pltpu.sync_copy(rows_v, out_hbm.at[pl.ds(base, _SC_CHUNK)])

    return gather_k(W, idx)


def kernel(z_e, W):
    zn = jnp.sum(z_e ** 2, axis=1, keepdims=True)
    cn = jnp.sum(W ** 2, axis=1).reshape(1, _NUM_CODES)
    idx, dsum = _distance_argmin(z_e, W, zn, cn)
    z_q_st = _sc_gather(W, idx)
    loss = (dsum[0, 0] / jnp.float32(_TOKENS * _DIM)) * jnp.float32(_COMMIT)
    return (z_q_st, idx, loss)
